# baseline (speedup 1.0000x reference)
import functools

import jax
import jax.numpy as jnp
from jax import lax
from jax.experimental import pallas as pl
from jax.experimental.pallas import tpu as pltpu
from jax.experimental.pallas import tpu_sc as plsc

N_NODES = 100000
N_EDGES = 3200000
D = 16

NC, NS = 2, 16
NW = NC * NS
BLK = 128
NBLK = N_EDGES // BLK
BAT = 4
BE = BAT * BLK
SEGW = BAT * BLK * 8
NBAT = NBLK // BAT
MAIN_Q = NBAT // NW
TAIL_Q = NBAT - MAIN_Q * NW
STRIPE = N_NODES // NS

_mesh = plsc.VectorSubcoreMesh(core_axis_name="c", subcore_axis_name="s")


@functools.partial(
    pl.kernel,
    out_type=[
        jax.ShapeDtypeStruct((N_NODES, D), jnp.float32),
        jax.ShapeDtypeStruct((N_NODES, D), jnp.float32),
    ],
    mesh=_mesh,
    compiler_params=pltpu.CompilerParams(use_tc_tiling_on_sc=False,
                                         needs_layout_passes=False),
    scratch_types=[
        pltpu.VMEM_SHARED((N_NODES, D), jnp.float32),
        pltpu.VMEM((2, BE), jnp.int32),
        pltpu.VMEM((2, 2 * SEGW), jnp.float32),
        pltpu.VMEM((BE, D), jnp.float32),
        pltpu.SemaphoreType.DMA,
        pltpu.SemaphoreType.DMA,
        pltpu.SemaphoreType.DMA,
        pltpu.SemaphoreType.DMA,
    ],
)
def _scatter_sc(attr_hbm, idx_hbm, out0, out1,
                acc, idx_v, stg_v, em_v, is0, is1, as0, as1):
    cid = lax.axis_index("c")
    sid = lax.axis_index("s")
    wid = sid * NC + cid
    isems = (is0, is1)
    asems = (as0, as1)

    def _z(i, c):
        em_v[i, :] = jnp.zeros((D,), jnp.float32)
        return c
    lax.fori_loop(0, BE, _z, 0)
    r0 = sid * STRIPE
    for kk in range(STRIPE // BE):
        pltpu.sync_copy(em_v, acc.at[pl.ds(r0 + kk * BE, BE)])
    rem = STRIPE - (STRIPE // BE) * BE
    pltpu.sync_copy(em_v.at[pl.ds(0, rem)],
                    acc.at[pl.ds(r0 + (STRIPE // BE) * BE, rem)])
    plsc.subcore_barrier()

    lane = lax.iota(jnp.int32, D)
    obase = (lane // 8) * SEGW + (lane % 8) * BLK

    def fire_load(s, q):
        t0 = q * BAT
        pltpu.async_copy(idx_hbm.at[pl.ds(q * BE, BE)],
                         idx_v.at[s], isems[s])
        pltpu.async_copy(attr_hbm.at[pl.ds(t0 * BLK * 8, SEGW)],
                         stg_v.at[s, pl.ds(0, SEGW)], asems[s])
        pltpu.async_copy(attr_hbm.at[pl.ds((NBLK + t0) * BLK * 8, SEGW)],
                         stg_v.at[s, pl.ds(SEGW, SEGW)], asems[s])

    def wait_load(s):
        pltpu.make_async_copy(idx_hbm.at[pl.ds(0, BE)],
                              idx_v.at[s], isems[s]).wait()
        pltpu.make_async_copy(attr_hbm.at[pl.ds(0, SEGW)],
                              stg_v.at[s, pl.ds(0, SEGW)], asems[s]).wait()
        pltpu.make_async_copy(attr_hbm.at[pl.ds(0, SEGW)],
                              stg_v.at[s, pl.ds(0, SEGW)], asems[s]).wait()

    def repack(s):
        for j in range(BAT):
            ob = obase + j * (BLK * 8)

            def _r(i, c, ob=ob, j=j):
                l0 = i * 16
                obi = ob + l0
                for u in range(16):
                    em_v[j * BLK + l0 + u, :] = plsc.load_gather(
                        stg_v.at[s], [obi + u])
                return c
            lax.fori_loop(0, BLK // 16, _r, 0)

    def do_batch(s):
        wait_load(s)
        repack(s)
        pltpu.sync_copy(em_v, acc.at[idx_v.at[s]], add=True)

    qb = wid * MAIN_Q

    fire_load(0, qb)
    fire_load(1, qb + 1)
    do_batch(0)

    def _pair(p, c):
        for g_off, s in ((1, 1), (2, 0)):
            g = 2 * p + g_off

            @pl.when(g + 1 < MAIN_Q)
            def _prefetch():
                fire_load(1 - s, qb + g + 1)

            do_batch(s)
        return c
    lax.fori_loop(0, (MAIN_Q - 1) // 2, _pair, 0)

    @pl.when(wid < TAIL_Q)
    def _tail():
        q = NW * MAIN_Q + wid
        fire_load(0, q)
        do_batch(0)

    plsc.subcore_barrier()

    @pl.when(cid == 0)
    def _w0():
        pltpu.sync_copy(acc.at[pl.ds(r0, STRIPE)], out0.at[pl.ds(r0, STRIPE)])

    @pl.when(cid == 1)
    def _w1():
        pltpu.sync_copy(acc.at[pl.ds(r0, STRIPE)], out1.at[pl.ds(r0, STRIPE)])


def _add_body(a_ref, b_ref, o_ref):
    o_ref[...] = a_ref[...] + b_ref[...]


def _tc_add(a, b):
    rows, cols = a.shape
    return pl.pallas_call(
        _add_body,
        out_shape=jax.ShapeDtypeStruct((rows, cols), jnp.float32),
    )(a, b)


def kernel(edge_attr, edge_index):
    v = (edge_attr.T.reshape(2, 8, N_EDGES // 128, 128)
         .transpose(0, 2, 1, 3).reshape(N_EDGES * D))
    idx1 = edge_index[1].astype(jnp.int32)
    p0, p1 = _scatter_sc(v, idx1)
    a = p0.reshape(N_NODES * D // 128, 128)
    b = p1.reshape(N_NODES * D // 128, 128)
    return _tc_add(a, b).reshape(N_NODES, D)


# speedup vs baseline: 2.4317x; 2.4317x over previous
import functools

import jax
import jax.numpy as jnp
from jax import lax
from jax.experimental import pallas as pl
from jax.experimental.pallas import tpu as pltpu
from jax.experimental.pallas import tpu_sc as plsc

N_NODES = 100000
N_EDGES = 3200000
D = 16

NC, NS = 2, 16
NW = NC * NS
BLK = 128
NBLK = N_EDGES // BLK
BAT = 4
BE = BAT * BLK
SEGW = BAT * BLK * 8
NBAT = NBLK // BAT
MAIN_Q = NBAT // NW
TAIL_Q = NBAT - MAIN_Q * NW
STRIPE = N_NODES // NS

_mesh = plsc.VectorSubcoreMesh(core_axis_name="c", subcore_axis_name="s")


@functools.partial(
    pl.kernel,
    out_type=[
        jax.ShapeDtypeStruct((N_NODES, D), jnp.float32),
        jax.ShapeDtypeStruct((N_NODES, D), jnp.float32),
    ],
    mesh=_mesh,
    compiler_params=pltpu.CompilerParams(use_tc_tiling_on_sc=False,
                                         needs_layout_passes=False),
    scratch_types=[
        pltpu.VMEM_SHARED((N_NODES, D), jnp.float32),
        pltpu.VMEM((2, BE), jnp.int32),
        pltpu.VMEM((2, 2 * SEGW), jnp.float32),
        pltpu.VMEM((BE, D), jnp.float32),
        pltpu.SemaphoreType.DMA,
        pltpu.SemaphoreType.DMA,
        pltpu.SemaphoreType.DMA,
        pltpu.SemaphoreType.DMA,
    ],
)
def _scatter_sc(attr_hbm, idx_hbm, out0, out1,
                acc, idx_v, stg_v, em_v, is0, is1, as0, as1):
    cid = lax.axis_index("c")
    sid = lax.axis_index("s")
    wid = sid * NC + cid
    isems = (is0, is1)
    asems = (as0, as1)

    def _z(i, c):
        em_v[i, :] = jnp.zeros((D,), jnp.float32)
        return c
    lax.fori_loop(0, BE, _z, 0)
    r0 = sid * STRIPE
    for kk in range(STRIPE // BE):
        pltpu.sync_copy(em_v, acc.at[pl.ds(r0 + kk * BE, BE)])
    rem = STRIPE - (STRIPE // BE) * BE
    pltpu.sync_copy(em_v.at[pl.ds(0, rem)],
                    acc.at[pl.ds(r0 + (STRIPE // BE) * BE, rem)])
    plsc.subcore_barrier()

    lane = lax.iota(jnp.int32, D)
    _col = [(lane + k) % D for k in range(D)]
    _ob = [(c // 8) * SEGW + (c % 8) * BLK for c in _col]

    def fire_load(s, q):
        t0 = q * BAT
        pltpu.async_copy(idx_hbm.at[pl.ds(q * BE, BE)],
                         idx_v.at[s], isems[s])
        pltpu.async_copy(attr_hbm.at[pl.ds(t0 * BLK * 8, SEGW)],
                         stg_v.at[s, pl.ds(0, SEGW)], asems[s])
        pltpu.async_copy(attr_hbm.at[pl.ds((NBLK + t0) * BLK * 8, SEGW)],
                         stg_v.at[s, pl.ds(SEGW, SEGW)], asems[s])

    def wait_load(s):
        pltpu.make_async_copy(idx_hbm.at[pl.ds(0, BE)],
                              idx_v.at[s], isems[s]).wait()
        pltpu.make_async_copy(attr_hbm.at[pl.ds(0, SEGW)],
                              stg_v.at[s, pl.ds(0, SEGW)], asems[s]).wait()
        pltpu.make_async_copy(attr_hbm.at[pl.ds(0, SEGW)],
                              stg_v.at[s, pl.ds(0, SEGW)], asems[s]).wait()

    def repack(s):
        for j in range(BAT):

            def _r(i, c, j=j):
                gbase = j * (BLK * 8) + i * 16
                rbase = j * BLK + i * 16
                gv = lane + gbase
                rv = lane + rbase
                for k in range(D):
                    val = plsc.load_gather(stg_v.at[s], [_ob[k] + gv])
                    plsc.store_scatter(em_v, [rv, _col[k]], val)
                return c
            lax.fori_loop(0, BLK // 16, _r, 0)

    def do_batch(s):
        wait_load(s)
        repack(s)
        pltpu.sync_copy(em_v, acc.at[idx_v.at[s]], add=True)

    qb = wid * MAIN_Q

    fire_load(0, qb)
    fire_load(1, qb + 1)
    do_batch(0)

    def _pair(p, c):
        for g_off, s in ((1, 1), (2, 0)):
            g = 2 * p + g_off

            @pl.when(g + 1 < MAIN_Q)
            def _prefetch():
                fire_load(1 - s, qb + g + 1)

            do_batch(s)
        return c
    lax.fori_loop(0, (MAIN_Q - 1) // 2, _pair, 0)

    @pl.when(wid < TAIL_Q)
    def _tail():
        q = NW * MAIN_Q + wid
        fire_load(0, q)
        do_batch(0)

    plsc.subcore_barrier()

    @pl.when(cid == 0)
    def _w0():
        pltpu.sync_copy(acc.at[pl.ds(r0, STRIPE)], out0.at[pl.ds(r0, STRIPE)])

    @pl.when(cid == 1)
    def _w1():
        pltpu.sync_copy(acc.at[pl.ds(r0, STRIPE)], out1.at[pl.ds(r0, STRIPE)])


def _add_body(a_ref, b_ref, o_ref):
    o_ref[...] = a_ref[...] + b_ref[...]


def _tc_add(a, b):
    rows, cols = a.shape
    return pl.pallas_call(
        _add_body,
        out_shape=jax.ShapeDtypeStruct((rows, cols), jnp.float32),
    )(a, b)


def kernel(edge_attr, edge_index):
    v = (edge_attr.T.reshape(2, 8, N_EDGES // 128, 128)
         .transpose(0, 2, 1, 3).reshape(N_EDGES * D))
    idx1 = edge_index[1].astype(jnp.int32)
    p0, p1 = _scatter_sc(v, idx1)
    a = p0.reshape(N_NODES * D // 128, 128)
    b = p1.reshape(N_NODES * D // 128, 128)
    return _tc_add(a, b).reshape(N_NODES, D)
